# baseline (speedup 1.0000x reference)
import functools

import jax
import jax.numpy as jnp
from jax import lax
from jax.experimental import pallas as pl
from jax.experimental.pallas import tpu as pltpu
from jax.experimental.pallas import tpu_sc as plsc


_GATHER_WINDOW = 256


def _sc_gather(table, idx_flat):
    n = idx_flat.shape[1]
    d = table.shape[1]
    mesh = plsc.VectorSubcoreMesh(core_axis_name="c", subcore_axis_name="s")

    @functools.partial(
        pl.kernel,
        out_type=jax.ShapeDtypeStruct((n, d), table.dtype),
        mesh=mesh,
    )
    def gather_kernel(table_hbm, idx_hbm, out_hbm):
        def body(idx_vmem, out_vmem):
            pltpu.sync_copy(table_hbm.at[idx_vmem.at[0]], out_vmem)

        pltpu.emit_pipeline(
            body,
            grid=(n // _GATHER_WINDOW,),
            in_specs=[
                pl.BlockSpec((1, _GATHER_WINDOW), index_map=lambda i: (0, i))
            ],
            out_specs=[
                pl.BlockSpec((_GATHER_WINDOW, d), index_map=lambda i: (i, 0))
            ],
            core_axis_name=("c", "s"),
            dimension_semantics=(pltpu.PARALLEL,),
        )(idx_hbm, out_hbm)

    return gather_kernel(table, idx_flat)


_M_TILE = 1024
_P_CHUNK = 8


def _head_kernel(nk, nm, tok_ref, pos_ref, w_ref, b_ref, truth_ref,
                 out_ref, loss_ref):
    k = pl.program_id(1)
    pc = tok_ref.shape[0]
    emb = jnp.concatenate(
        [(tok_ref[j] + pos_ref[j]).astype(jnp.bfloat16) for j in range(pc)],
        axis=1)
    wmat = w_ref[...].reshape(pc * w_ref.shape[1], w_ref.shape[2])
    part = lax.dot_general(
        emb, wmat, (((1,), (0,)), ((), ())),
        preferred_element_type=jnp.float32)

    @pl.when(k == 0)
    def _():
        out_ref[...] = part

    @pl.when(k > 0)
    def _():
        out_ref[...] += part

    @pl.when(k == nk - 1)
    def _():
        logits = out_ref[...] + b_ref[0:1, :]
        out_ref[...] = logits
        mt, v = logits.shape
        mx = jnp.max(logits, axis=1, keepdims=True)
        lse = mx + jnp.log(jnp.sum(jnp.exp(logits - mx), axis=1, keepdims=True))
        lane = lax.broadcasted_iota(jnp.int32, (mt, v), 1)
        tl = jnp.sum(
            jnp.where(lane == truth_ref[...], logits, 0.0),
            axis=1, keepdims=True)
        part_loss = jnp.sum(lse - tl)
        m = pl.program_id(0)

        @pl.when(m == 0)
        def _():
            loss_ref[0, 0] = part_loss

        @pl.when(m > 0)
        def _():
            loss_ref[0, 0] += part_loss


def _head(tok3, pos, w3, b_b, truth2d):
    l, cb, d = tok3.shape
    v = w3.shape[2]
    mt = min(cb, _M_TILE)
    nm = cb // mt
    nk = l // _P_CHUNK
    out, loss = pl.pallas_call(
        functools.partial(_head_kernel, nk, nm),
        grid=(nm, nk),
        in_specs=[
            pl.BlockSpec((_P_CHUNK, mt, d), lambda m, k: (k, m, 0)),
            pl.BlockSpec((_P_CHUNK, d), lambda m, k: (k, 0)),
            pl.BlockSpec((_P_CHUNK, d, v), lambda m, k: (k, 0, 0)),
            pl.BlockSpec((8, v), lambda m, k: (0, 0)),
            pl.BlockSpec((mt, 1), lambda m, k: (m, 0)),
        ],
        out_specs=[
            pl.BlockSpec((mt, v), lambda m, k: (m, 0)),
            pl.BlockSpec(
                (1, 1), lambda m, k: (0, 0), memory_space=pltpu.SMEM),
        ],
        out_shape=[
            jax.ShapeDtypeStruct((cb, v), jnp.float32),
            jax.ShapeDtypeStruct((1, 1), jnp.float32),
        ],
    )(tok3, pos, w3, b_b, truth2d)
    return out, loss


_CHUNK_SIZES = (512, 512, 1024, 2048)


def kernel(input_tokens, truth, tok_table, pos_table, W, b):
    bsz, l = input_tokens.shape
    v, d = tok_table.shape
    w3 = W.astype(jnp.bfloat16).reshape(v, l, d).transpose(1, 2, 0)
    b_b = jnp.broadcast_to(b.reshape(1, v), (8, v))
    idx = input_tokens.astype(jnp.int32)
    truth2d = truth.reshape(bsz, 1).astype(jnp.int32)
    offs = [0]
    for cs in _CHUNK_SIZES:
        offs.append(offs[-1] + cs)
    tok3s = [
        _sc_gather(
            tok_table,
            idx[offs[c]:offs[c + 1]].T.reshape(1, l * cs)).reshape(l, cs, d)
        for c, cs in enumerate(_CHUNK_SIZES)
    ]
    outs, loss_sums = [], []
    for c, cs in enumerate(_CHUNK_SIZES):
        out_c, loss_c = _head(
            tok3s[c], pos_table, w3, b_b, truth2d[offs[c]:offs[c + 1]])
        outs.append(out_c)
        loss_sums.append(loss_c[0, 0])
    out = jnp.concatenate(outs, axis=0)
    loss = sum(loss_sums) / bsz
    return out, loss.reshape(())


# speedup vs baseline: 1.0394x; 1.0394x over previous
import functools

import jax
import jax.numpy as jnp
from jax import lax
from jax.experimental import pallas as pl
from jax.experimental.pallas import tpu as pltpu
from jax.experimental.pallas import tpu_sc as plsc


_GATHER_WINDOW = 256


def _sc_gather(table, idx_flat):
    n = idx_flat.shape[1]
    d = table.shape[1]
    mesh = plsc.VectorSubcoreMesh(core_axis_name="c", subcore_axis_name="s")

    @functools.partial(
        pl.kernel,
        out_type=jax.ShapeDtypeStruct((n, d), table.dtype),
        mesh=mesh,
    )
    def gather_kernel(table_hbm, idx_hbm, out_hbm):
        def body(idx_vmem, out_vmem):
            pltpu.sync_copy(table_hbm.at[idx_vmem.at[0]], out_vmem)

        pltpu.emit_pipeline(
            body,
            grid=(n // _GATHER_WINDOW,),
            in_specs=[
                pl.BlockSpec((1, _GATHER_WINDOW), index_map=lambda i: (0, i))
            ],
            out_specs=[
                pl.BlockSpec((_GATHER_WINDOW, d), index_map=lambda i: (i, 0))
            ],
            core_axis_name=("c", "s"),
            dimension_semantics=(pltpu.PARALLEL,),
        )(idx_hbm, out_hbm)

    return gather_kernel(table, idx_flat)


_M_TILE = 1024
_P_CHUNK = 25


def _beff_kernel(nk, pos_ref, w_ref, b_ref, out_ref):
    k = pl.program_id(0)
    pc = pos_ref.shape[0]
    emb = jnp.concatenate(
        [pos_ref[j].astype(jnp.bfloat16) for j in range(pc)], axis=1)
    wmat = w_ref[...].reshape(pc * w_ref.shape[1], w_ref.shape[2])
    part = lax.dot_general(
        emb, wmat, (((1,), (0,)), ((), ())),
        preferred_element_type=jnp.float32)

    @pl.when(k == 0)
    def _():
        out_ref[...] = b_ref[0:1, :] + part

    @pl.when(k > 0)
    def _():
        out_ref[...] += part


def _beff(pos3, w3, b_b):
    l, d, v = w3.shape
    nk = l // _P_CHUNK
    return pl.pallas_call(
        functools.partial(_beff_kernel, nk),
        grid=(nk,),
        in_specs=[
            pl.BlockSpec((_P_CHUNK, 1, d), lambda k: (k, 0, 0)),
            pl.BlockSpec((_P_CHUNK, d, v), lambda k: (k, 0, 0)),
            pl.BlockSpec((8, v), lambda k: (0, 0)),
        ],
        out_specs=pl.BlockSpec((1, v), lambda k: (0, 0)),
        out_shape=jax.ShapeDtypeStruct((1, v), jnp.float32),
    )(pos3, w3, b_b)


def _head_kernel(nk, nm, tok_ref, w_ref, b_ref, truth_ref,
                 out_ref, loss_ref):
    k = pl.program_id(1)
    pc = tok_ref.shape[0]
    emb = jnp.concatenate(
        [tok_ref[j].astype(jnp.bfloat16) for j in range(pc)], axis=1)
    wmat = w_ref[...].reshape(pc * w_ref.shape[1], w_ref.shape[2])
    part = lax.dot_general(
        emb, wmat, (((1,), (0,)), ((), ())),
        preferred_element_type=jnp.float32)

    @pl.when(k == 0)
    def _():
        out_ref[...] = part

    @pl.when(k > 0)
    def _():
        out_ref[...] += part

    @pl.when(k == nk - 1)
    def _():
        logits = out_ref[...] + b_ref[0:1, :]
        out_ref[...] = logits
        mt, v = logits.shape
        mx = jnp.max(logits, axis=1, keepdims=True)
        lse = mx + jnp.log(jnp.sum(jnp.exp(logits - mx), axis=1, keepdims=True))
        lane = lax.broadcasted_iota(jnp.int32, (mt, v), 1)
        tl = jnp.sum(
            jnp.where(lane == truth_ref[...], logits, 0.0),
            axis=1, keepdims=True)
        part_loss = jnp.sum(lse - tl)
        m = pl.program_id(0)

        @pl.when(m == 0)
        def _():
            loss_ref[0, 0] = part_loss

        @pl.when(m > 0)
        def _():
            loss_ref[0, 0] += part_loss


def _head(tok3, w3, b_eff, truth2d):
    l, cb, d = tok3.shape
    v = w3.shape[2]
    mt = min(cb, _M_TILE)
    nm = cb // mt
    nk = l // _P_CHUNK
    out, loss = pl.pallas_call(
        functools.partial(_head_kernel, nk, nm),
        grid=(nm, nk),
        in_specs=[
            pl.BlockSpec((_P_CHUNK, mt, d), lambda m, k: (k, m, 0)),
            pl.BlockSpec((_P_CHUNK, d, v), lambda m, k: (k, 0, 0)),
            pl.BlockSpec((1, v), lambda m, k: (0, 0)),
            pl.BlockSpec((mt, 1), lambda m, k: (m, 0)),
        ],
        out_specs=[
            pl.BlockSpec((mt, v), lambda m, k: (m, 0)),
            pl.BlockSpec(
                (1, 1), lambda m, k: (0, 0), memory_space=pltpu.SMEM),
        ],
        out_shape=[
            jax.ShapeDtypeStruct((cb, v), jnp.float32),
            jax.ShapeDtypeStruct((1, 1), jnp.float32),
        ],
    )(tok3, w3, b_eff, truth2d)
    return out, loss


_CHUNK_SIZES = (1024, 1024, 1024, 1024)


def kernel(input_tokens, truth, tok_table, pos_table, W, b):
    bsz, l = input_tokens.shape
    v, d = tok_table.shape
    w3 = W.astype(jnp.bfloat16).reshape(v, l, d).transpose(1, 2, 0)
    b_b = jnp.broadcast_to(b.reshape(1, v), (8, v))
    idx = input_tokens.astype(jnp.int32)
    truth2d = truth.reshape(bsz, 1).astype(jnp.int32)
    offs = [0]
    for cs in _CHUNK_SIZES:
        offs.append(offs[-1] + cs)
    tok3s = [
        _sc_gather(
            tok_table,
            idx[offs[c]:offs[c + 1]].T.reshape(1, l * cs)).reshape(l, cs, d)
        for c, cs in enumerate(_CHUNK_SIZES)
    ]
    b_eff = _beff(pos_table.reshape(l, 1, d), w3, b_b)
    outs, loss_sums = [], []
    for c, cs in enumerate(_CHUNK_SIZES):
        out_c, loss_c = _head(
            tok3s[c], w3, b_eff, truth2d[offs[c]:offs[c + 1]])
        outs.append(out_c)
        loss_sums.append(loss_c[0, 0])
    out = jnp.concatenate(outs, axis=0)
    loss = sum(loss_sums) / bsz
    return out, loss.reshape(())


# speedup vs baseline: 1.0434x; 1.0038x over previous
import functools

import jax
import jax.numpy as jnp
from jax import lax
from jax.experimental import pallas as pl
from jax.experimental.pallas import tpu as pltpu
from jax.experimental.pallas import tpu_sc as plsc


_GATHER_WINDOW = 160
_N_BUF = 4
_N_WORKERS = 32


def _sc_gather(table, idx_flat):
    n = idx_flat.shape[1]
    d = table.shape[1]
    w = _GATHER_WINDOW
    rpw = n // _N_WORKERS
    ngroups = rpw // (w * _N_BUF)
    mesh = plsc.VectorSubcoreMesh(core_axis_name="c", subcore_axis_name="s")

    @functools.partial(
        pl.kernel,
        out_type=jax.ShapeDtypeStruct((n, d), table.dtype),
        mesh=mesh,
        scratch_types=(
            [pltpu.VMEM((w, d), table.dtype) for _ in range(_N_BUF)]
            + [
                pltpu.VMEM((rpw,), jnp.int32),
                pltpu.SemaphoreType.DMA,
                pltpu.SemaphoreType.DMA,
                pltpu.SemaphoreType.DMA,
            ]
        ),
    )
    def gather_kernel(table_hbm, idx_hbm, out_hbm, *scratch):
        rows = scratch[:_N_BUF]
        idx_v, i_sem, g_sem, o_sem = scratch[_N_BUF:]
        wid = jax.lax.axis_index("s") * 2 + jax.lax.axis_index("c")
        base = wid * rpw
        pltpu.async_copy(idx_hbm.at[0, pl.ds(base, rpw)], idx_v, i_sem).wait()

        def gather_start(b, win):
            pltpu.async_copy(
                table_hbm.at[idx_v.at[pl.ds(win * w, w)]], rows[b], g_sem)

        def gather_wait(b, win):
            pltpu.make_async_copy(
                table_hbm.at[idx_v.at[pl.ds(win * w, w)]], rows[b],
                g_sem).wait()

        def out_start(b, win):
            pltpu.async_copy(
                rows[b], out_hbm.at[pl.ds(base + win * w, w)], o_sem)

        def out_wait(b, win):
            pltpu.make_async_copy(
                rows[b], out_hbm.at[pl.ds(base + win * w, w)], o_sem).wait()

        for b in range(_N_BUF):
            gather_start(b, b)

        @pl.loop(0, ngroups)
        def _(g):
            for b in range(_N_BUF):
                win = g * _N_BUF + b
                gather_wait(b, win)
                out_start(b, win)
                nxt = win + _N_BUF

                @pl.when(g < ngroups - 1)
                def _():
                    out_wait(b, win)
                    gather_start(b, nxt)

        for b in range(_N_BUF):
            out_wait(b, (ngroups - 1) * _N_BUF + b)

    return gather_kernel(table, idx_flat)


_M_TILE = 1024
_P_CHUNK = 25


def _beff_kernel(nk, pos_ref, w_ref, b_ref, out_ref):
    k = pl.program_id(0)
    pc = pos_ref.shape[0]
    emb = jnp.concatenate(
        [pos_ref[j].astype(jnp.bfloat16) for j in range(pc)], axis=1)
    wmat = w_ref[...].reshape(pc * w_ref.shape[1], w_ref.shape[2])
    part = lax.dot_general(
        emb, wmat, (((1,), (0,)), ((), ())),
        preferred_element_type=jnp.float32)

    @pl.when(k == 0)
    def _():
        out_ref[...] = b_ref[0:1, :] + part

    @pl.when(k > 0)
    def _():
        out_ref[...] += part


def _beff(pos3, w3, b_b):
    l, d, v = w3.shape
    nk = l // _P_CHUNK
    return pl.pallas_call(
        functools.partial(_beff_kernel, nk),
        grid=(nk,),
        in_specs=[
            pl.BlockSpec((_P_CHUNK, 1, d), lambda k: (k, 0, 0)),
            pl.BlockSpec((_P_CHUNK, d, v), lambda k: (k, 0, 0)),
            pl.BlockSpec((8, v), lambda k: (0, 0)),
        ],
        out_specs=pl.BlockSpec((1, v), lambda k: (0, 0)),
        out_shape=jax.ShapeDtypeStruct((1, v), jnp.float32),
    )(pos3, w3, b_b)


def _head_kernel(nk, nm, tok_ref, w_ref, b_ref, truth_ref,
                 out_ref, loss_ref):
    k = pl.program_id(1)
    pc = tok_ref.shape[0]
    emb = jnp.concatenate(
        [tok_ref[j].astype(jnp.bfloat16) for j in range(pc)], axis=1)
    wmat = w_ref[...].reshape(pc * w_ref.shape[1], w_ref.shape[2])
    part = lax.dot_general(
        emb, wmat, (((1,), (0,)), ((), ())),
        preferred_element_type=jnp.float32)

    @pl.when(k == 0)
    def _():
        out_ref[...] = part

    @pl.when(k > 0)
    def _():
        out_ref[...] += part

    @pl.when(k == nk - 1)
    def _():
        logits = out_ref[...] + b_ref[0:1, :]
        out_ref[...] = logits
        mt, v = logits.shape
        mx = jnp.max(logits, axis=1, keepdims=True)
        lse = mx + jnp.log(jnp.sum(jnp.exp(logits - mx), axis=1, keepdims=True))
        lane = lax.broadcasted_iota(jnp.int32, (mt, v), 1)
        tl = jnp.sum(
            jnp.where(lane == truth_ref[...], logits, 0.0),
            axis=1, keepdims=True)
        part_loss = jnp.sum(lse - tl)
        m = pl.program_id(0)

        @pl.when(m == 0)
        def _():
            loss_ref[0, 0] = part_loss

        @pl.when(m > 0)
        def _():
            loss_ref[0, 0] += part_loss


def _head(tok3, w3, b_eff, truth2d):
    l, cb, d = tok3.shape
    v = w3.shape[2]
    mt = min(cb, _M_TILE)
    nm = cb // mt
    nk = l // _P_CHUNK
    out, loss = pl.pallas_call(
        functools.partial(_head_kernel, nk, nm),
        grid=(nm, nk),
        in_specs=[
            pl.BlockSpec((_P_CHUNK, mt, d), lambda m, k: (k, m, 0)),
            pl.BlockSpec((_P_CHUNK, d, v), lambda m, k: (k, 0, 0)),
            pl.BlockSpec((1, v), lambda m, k: (0, 0)),
            pl.BlockSpec((mt, 1), lambda m, k: (m, 0)),
        ],
        out_specs=[
            pl.BlockSpec((mt, v), lambda m, k: (m, 0)),
            pl.BlockSpec(
                (1, 1), lambda m, k: (0, 0), memory_space=pltpu.SMEM),
        ],
        out_shape=[
            jax.ShapeDtypeStruct((cb, v), jnp.float32),
            jax.ShapeDtypeStruct((1, 1), jnp.float32),
        ],
    )(tok3, w3, b_eff, truth2d)
    return out, loss


_CHUNK_SIZES = (1024, 1024, 1024, 1024)


def kernel(input_tokens, truth, tok_table, pos_table, W, b):
    bsz, l = input_tokens.shape
    v, d = tok_table.shape
    w3 = W.astype(jnp.bfloat16).reshape(v, l, d).transpose(1, 2, 0)
    b_b = jnp.broadcast_to(b.reshape(1, v), (8, v))
    idx = input_tokens.astype(jnp.int32)
    truth2d = truth.reshape(bsz, 1).astype(jnp.int32)
    offs = [0]
    for cs in _CHUNK_SIZES:
        offs.append(offs[-1] + cs)
    tok3s = [
        _sc_gather(
            tok_table,
            idx[offs[c]:offs[c + 1]].T.reshape(1, l * cs)).reshape(l, cs, d)
        for c, cs in enumerate(_CHUNK_SIZES)
    ]
    b_eff = _beff(pos_table.reshape(l, 1, d), w3, b_b)
    outs, loss_sums = [], []
    for c, cs in enumerate(_CHUNK_SIZES):
        out_c, loss_c = _head(
            tok3s[c], w3, b_eff, truth2d[offs[c]:offs[c + 1]])
        outs.append(out_c)
        loss_sums.append(loss_c[0, 0])
    out = jnp.concatenate(outs, axis=0)
    loss = sum(loss_sums) / bsz
    return out, loss.reshape(())
